# baseline (speedup 1.0000x reference)
import functools

import jax
import jax.numpy as jnp
from jax import lax
from jax.experimental import pallas as pl
from jax.experimental.pallas import tpu as pltpu
from jax.experimental.pallas import tpu_sc as plsc

D = 768
N = 4096
NW = 32
N_PER_W = N // NW
CHUNK = 16
N_CHUNKS = N_PER_W // CHUNK
NB = 2
GROUPS = N_CHUNKS // NB
LANES = 16


def _sc_embed_add(ids, hidden, table):
    mesh = plsc.VectorSubcoreMesh(core_axis_name="c", subcore_axis_name="s")

    scratch = [pltpu.VMEM((N_PER_W,), jnp.int32)]
    scratch += [pltpu.VMEM((CHUNK, D), jnp.float32) for _ in range(3 * NB)]
    scratch += [pltpu.SemaphoreType.DMA for _ in range(3 * NB)]

    @functools.partial(
        pl.kernel,
        mesh=mesh,
        out_type=jax.ShapeDtypeStruct((N, D), jnp.float32),
        scratch_types=scratch,
    )
    def k(ids_hbm, hid_hbm, table_hbm, out_hbm, idx_v, *bufs):
        rows = bufs[0:NB]
        hid = bufs[NB:2 * NB]
        obuf = bufs[2 * NB:3 * NB]
        gsem = bufs[3 * NB:4 * NB]
        hsem = bufs[4 * NB:5 * NB]
        osem = bufs[5 * NB:6 * NB]

        wid = lax.axis_index("s") * 2 + lax.axis_index("c")
        base = wid * N_PER_W
        pltpu.sync_copy(ids_hbm.at[pl.ds(base, N_PER_W)], idx_v)

        def mk_g(c, b):
            return pltpu.make_async_copy(
                table_hbm.at[idx_v.at[pl.ds(c * CHUNK, CHUNK)]], rows[b], gsem[b]
            )

        def mk_h(c, b):
            return pltpu.make_async_copy(
                hid_hbm.at[pl.ds(base + c * CHUNK, CHUNK)], hid[b], hsem[b]
            )

        def mk_o(c, b):
            return pltpu.make_async_copy(
                obuf[b], out_hbm.at[pl.ds(base + c * CHUNK, CHUNK)], osem[b]
            )

        for b in range(NB):
            mk_g(b, b).start()
            mk_h(b, b).start()

        def group(g, carry):
            for b in range(NB):
                c = g * NB + b
                mk_g(c, b).wait()
                mk_h(c, b).wait()

                @pl.when(g > 0)
                def _():
                    mk_o(c - NB, b).wait()

                def add_row(i, carry2):
                    for j in range(D // LANES):
                        sl = pl.ds(j * LANES, LANES)
                        obuf[b][i, sl] = rows[b][i, sl] + hid[b][i, sl]
                    return carry2

                lax.fori_loop(0, CHUNK, add_row, 0)

                @pl.when(g < GROUPS - 1)
                def _():
                    mk_g(c + NB, b).start()
                    mk_h(c + NB, b).start()

                mk_o(c, b).start()
            return carry

        lax.fori_loop(0, GROUPS, group, 0)
        for b in range(NB):
            mk_o((GROUPS - 1) * NB + b, b).wait()

    return k(ids, hidden, table)


def kernel(input_ids, hidden_states, positions, embed_table):
    ids = input_ids.reshape(-1).astype(jnp.int32)
    hid = hidden_states.reshape(N, D)
    out = _sc_embed_add(ids, hid, embed_table)
    return out.reshape(hidden_states.shape)
